# baseline (speedup 1.0000x reference)
import functools

import jax
import jax.numpy as jnp
import numpy as np
from jax import lax
from jax.experimental import pallas as pl
from jax.experimental.pallas import tpu as pltpu
from jax.experimental.pallas import tpu_sc as plsc

_N = 10000
_E = 320000
_D = 128
_H = 8
_DH = 16
_SCALE = 1.0 / np.sqrt(_DH)

_NW = 32
_EPW = _E // _NW
_B = 80
_NCHUNK = _EPW // _B
_BLK = 2000
_SH = 90112
_ZCH = 704

_EXPAND = np.kron(np.eye(_H), np.ones((1, _DH))).astype(np.float32)


def _proj_body(x_ref, wq_ref, wk_ref, wv_ref, q_ref, k_ref, v_ref):
    xb = x_ref[...]
    dn = (((1,), (1,)), ((), ()))
    q_ref[...] = lax.dot_general(xb, wq_ref[...], dn,
                                 preferred_element_type=jnp.float32)
    k_ref[...] = lax.dot_general(xb, wk_ref[...], dn,
                                 preferred_element_type=jnp.float32)
    v_ref[...] = lax.dot_general(xb, wv_ref[...], dn,
                                 preferred_element_type=jnp.float32)


def _projections(x, w_q, w_k, w_v):
    blk = 1000
    grid = _N // blk
    out = jax.ShapeDtypeStruct((_N, _D), jnp.float32)
    return pl.pallas_call(
        _proj_body,
        grid=(grid,),
        in_specs=[
            pl.BlockSpec((blk, _D), lambda i: (i, 0)),
            pl.BlockSpec((_D, _D), lambda i: (0, 0)),
            pl.BlockSpec((_D, _D), lambda i: (0, 0)),
            pl.BlockSpec((_D, _D), lambda i: (0, 0)),
        ],
        out_specs=[
            pl.BlockSpec((blk, _D), lambda i: (i, 0)),
            pl.BlockSpec((blk, _D), lambda i: (i, 0)),
            pl.BlockSpec((blk, _D), lambda i: (i, 0)),
        ],
        out_shape=[out, out, out],
    )(x, w_q, w_k, w_v)


def _sc_scores(Q, K, src, dst, edge_attr):
    mesh = plsc.VectorSubcoreMesh(core_axis_name="c", subcore_axis_name="s")

    @functools.partial(
        pl.kernel,
        out_type=jax.ShapeDtypeStruct((2, _SH), jnp.float32),
        mesh=mesh,
        scratch_types=[
            pltpu.VMEM_SHARED((_SH,), jnp.float32),
            pltpu.VMEM((_BLK,), jnp.int32),
            pltpu.VMEM((_BLK,), jnp.int32),
            pltpu.VMEM((_BLK,), jnp.float32),
            pltpu.VMEM((400, 4), jnp.float32),
            pltpu.VMEM((_B, _D), jnp.float32),
            pltpu.VMEM((_B, _D), jnp.float32),
            pltpu.VMEM((_B, _D), jnp.float32),
            pltpu.VMEM((_B, _D), jnp.float32),
            pltpu.VMEM((720,), jnp.float32),
            pltpu.VMEM((720,), jnp.float32),
            pltpu.VMEM((720,), jnp.int32),
            pltpu.VMEM((720,), jnp.int32),
            pltpu.VMEM((_ZCH,), jnp.float32),
            pltpu.SemaphoreType.DMA,
            pltpu.SemaphoreType.DMA,
            pltpu.SemaphoreType.DMA,
            pltpu.SemaphoreType.DMA,
        ],
        compiler_params=pltpu.CompilerParams(needs_layout_passes=False),
    )
    def body(q_hbm, k_hbm, src_hbm, dst_hbm, ea_hbm, out_hbm,
             shared, src2k, dst2k, gate2k, eatmp,
             qb0, qb1, kb0, kb1, sb0, sb1, ib0, ib1, zbuf,
             semr0, semr1, sems0, sems1):
        cidx = lax.axis_index("c")
        sidx = lax.axis_index("s")
        wid = sidx * 2 + cidx
        lanes = lax.iota(jnp.int32, 16)
        rot = [jnp.full((16,), d, jnp.int32) for d in range(16)]
        zeros16 = jnp.zeros((16,), jnp.float32)
        ones16 = jnp.full((16,), 1.0, jnp.float32)

        qb = (qb0, qb1)
        kb = (kb0, kb1)
        sb = (sb0, sb1)
        ib = (ib0, ib1)
        semr = (semr0, semr1)
        sems = (sems0, sems1)

        for j in range(_ZCH // 16):
            zbuf[pl.ds(j * 16, 16)] = zeros16
        zbase = sidx * (_ZCH * 8)
        for j in range(8):
            pltpu.sync_copy(zbuf, shared.at[pl.ds(zbase + j * _ZCH, _ZCH)])
        plsc.subcore_barrier()

        ebase = wid * _EPW

        def stage_block(abase):
            abase = pl.multiple_of(abase, _B)
            pltpu.sync_copy(src_hbm.at[pl.ds(abase, _BLK)], src2k)
            pltpu.sync_copy(dst_hbm.at[pl.ds(abase, _BLK)], dst2k)
            for p in range(5):
                pltpu.sync_copy(ea_hbm.at[pl.ds(abase + p * 400, 400)], eatmp)

                def gate_body(j, c):
                    joff = pl.multiple_of(j * 16, 16)
                    rowv = j * 16 + lanes
                    asum = zeros16
                    for a in range(4):
                        ac = jnp.full((16,), a, jnp.int32)
                        asum = asum + plsc.load_gather(eatmp, [rowv, ac])
                    gate2k[pl.ds(p * 400 + joff, 16)] = \
                        _SCALE / (1.0 + jnp.exp(-asum))
                    return c

                lax.fori_loop(0, 25, gate_body, 0)

        def issue_rows(off, b):
            off = pl.multiple_of(off, _B)
            pltpu.async_copy(q_hbm.at[src2k.at[pl.ds(off, _B)]], qb[b],
                             semr[b])
            pltpu.async_copy(k_hbm.at[dst2k.at[pl.ds(off, _B)]], kb[b],
                             semr[b])

        def wait_rows(off, b):
            off = pl.multiple_of(off, _B)
            pltpu.make_async_copy(q_hbm.at[src2k.at[pl.ds(off, _B)]], qb[b],
                                  semr[b]).wait()
            pltpu.make_async_copy(k_hbm.at[dst2k.at[pl.ds(off, _B)]], kb[b],
                                  semr[b]).wait()

        def wait_scatter(b):
            pltpu.make_async_copy(sb[b], shared.at[ib[b]], sems[b]).wait()

        def compute_chunk(boff, b):
            qrows = qb[b]
            krows = kb[b]
            scores = sb[b]
            sidxb = ib[b]

            boff16 = pl.multiple_of(boff, 16)

            def group_body(g, c):
                go = pl.multiple_of(g * 16, 16)
                rowv = go + lanes
                dstg = dst2k[pl.ds(boff16 + go, 16)]
                wg = gate2k[pl.ds(boff16 + go, 16)]
                s9i = dstg * 9
                p0 = pl.multiple_of(g * 144, 16)
                for h in range(_H):
                    acc = zeros16
                    hbase = h * 16
                    for d in range(16):
                        colv = rot[d] + hbase
                        qv = plsc.load_gather(qrows, [rowv, colv])
                        kv = plsc.load_gather(krows, [rowv, colv])
                        acc = acc + qv * kv
                    scores[pl.ds(p0 + h * 16, 16)] = acc * wg
                    sidxb[pl.ds(p0 + h * 16, 16)] = s9i + h
                scores[pl.ds(p0 + 128, 16)] = ones16
                sidxb[pl.ds(p0 + 128, 16)] = s9i + 8
                return c

            lax.fori_loop(0, _B // 16, group_body, 0)

        stage_block(ebase)
        issue_rows(0, 0)

        def outer(t, carry):
            o_cur, a_cur = carry
            for b in range(2):
                o_next = jnp.where(o_cur + _B == _BLK, 0, o_cur + _B)
                a_next = a_cur + _B
                wait_rows(o_cur, b)

                @pl.when(o_next != 0)
                def _():
                    issue_rows(o_next, 1 - b)

                @pl.when(a_cur >= ebase + 2 * _B)
                def _():
                    wait_scatter(b)

                compute_chunk(o_cur, b)
                pltpu.async_copy(sb[b], shared.at[ib[b]], sems[b], add=True)

                @pl.when(o_next == 0)
                def _():
                    stage_block(a_next)
                    issue_rows(o_next, 1 - b)

                o_cur, a_cur = o_next, a_next
            return (o_cur, a_cur)

        o_l, a_l = lax.fori_loop(0, (_NCHUNK - 1) // 2, outer,
                                 (jnp.int32(0), ebase))

        wait_rows(o_l, 0)
        wait_scatter(0)
        compute_chunk(o_l, 0)
        pltpu.async_copy(sb[0], shared.at[ib[0]], sems[0], add=True)
        wait_scatter(1)
        wait_scatter(0)
        plsc.subcore_barrier()

        @pl.when(sidx == 0)
        def _():
            pltpu.sync_copy(shared, out_hbm.at[cidx])

    return body(Q, K, src, dst, edge_attr)


def _combine_body(sp_ref, v_ref, wo_ref, exp_ref, bo_ref, out_ref):
    s_all = jnp.sum(sp_ref[...], axis=0)
    cnt = s_all[:, 8:9]
    s = s_all[:, 0:8]
    m = jnp.where(cnt > 0, s / jnp.maximum(cnt, 1.0), 0.0)
    m128 = lax.dot_general(m, exp_ref[...], (((1,), (0,)), ((), ())),
                           preferred_element_type=jnp.float32)
    dn = (((1,), (1,)), ((), ()))
    out_ref[...] = lax.dot_general(v_ref[...] * m128, wo_ref[...], dn,
                                   preferred_element_type=jnp.float32) \
        + bo_ref[...]


def _combine(sp, V, w_o, b_o):
    blk = 1000
    grid = _N // blk
    return pl.pallas_call(
        _combine_body,
        grid=(grid,),
        in_specs=[
            pl.BlockSpec((2, blk, 9), lambda i: (0, i, 0)),
            pl.BlockSpec((blk, _D), lambda i: (i, 0)),
            pl.BlockSpec((_D, _D), lambda i: (0, 0)),
            pl.BlockSpec((_H, _D), lambda i: (0, 0)),
            pl.BlockSpec((1, _D), lambda i: (0, 0)),
        ],
        out_specs=pl.BlockSpec((blk, _D), lambda i: (i, 0)),
        out_shape=jax.ShapeDtypeStruct((_N, _D), jnp.float32),
    )(sp, V, w_o, _EXPAND, b_o)


def kernel(x, edge_index, edge_attr, w_q, w_k, w_v, w_o, b_o):
    Q, K, V = _projections(x, w_q, w_k, w_v)
    src = edge_index[0]
    dst = edge_index[1]
    sp = _sc_scores(Q, K, src, dst, edge_attr)
    sp9 = sp[:, :9 * _N].reshape(2, _N, 9)
    return _combine(sp9, V, w_o, b_o.reshape(1, _D))


# speedup vs baseline: 3.4128x; 3.4128x over previous
import functools

import jax
import jax.numpy as jnp
import numpy as np
from jax import lax
from jax.experimental import pallas as pl
from jax.experimental.pallas import tpu as pltpu
from jax.experimental.pallas import tpu_sc as plsc

_N = 10000
_E = 320000
_D = 128
_H = 8
_DH = 16
_SCALE = 1.0 / np.sqrt(_DH)

_NW = 32
_EPW = _E // _NW
_B = 80
_NCHUNK = _EPW // _B
_BLK = 2000
_SH = 90112
_ZCH = 704

_EXPAND = np.kron(np.eye(_H), np.ones((1, _DH))).astype(np.float32)


def _proj_body(x_ref, wq_ref, wk_ref, wv_ref, q_ref, k_ref, v_ref):
    xb = x_ref[...]
    dn = (((1,), (1,)), ((), ()))
    q_ref[...] = lax.dot_general(xb, wq_ref[...], dn,
                                 preferred_element_type=jnp.float32)
    k_ref[...] = lax.dot_general(xb, wk_ref[...], dn,
                                 preferred_element_type=jnp.float32)
    v_ref[...] = lax.dot_general(xb, wv_ref[...], dn,
                                 preferred_element_type=jnp.float32)


def _projections(x, w_q, w_k, w_v):
    blk = 1000
    grid = _N // blk
    out = jax.ShapeDtypeStruct((_N, _D), jnp.float32)
    return pl.pallas_call(
        _proj_body,
        grid=(grid,),
        in_specs=[
            pl.BlockSpec((blk, _D), lambda i: (i, 0)),
            pl.BlockSpec((_D, _D), lambda i: (0, 0)),
            pl.BlockSpec((_D, _D), lambda i: (0, 0)),
            pl.BlockSpec((_D, _D), lambda i: (0, 0)),
        ],
        out_specs=[
            pl.BlockSpec((blk, _D), lambda i: (i, 0)),
            pl.BlockSpec((blk, _D), lambda i: (i, 0)),
            pl.BlockSpec((blk, _D), lambda i: (i, 0)),
        ],
        out_shape=[out, out, out],
    )(x, w_q, w_k, w_v)


def _sc_scores(Q, K, src, dst, edge_attr):
    mesh = plsc.VectorSubcoreMesh(core_axis_name="c", subcore_axis_name="s")

    @functools.partial(
        pl.kernel,
        out_type=jax.ShapeDtypeStruct((2, _SH), jnp.float32),
        mesh=mesh,
        scratch_types=[
            pltpu.VMEM_SHARED((_SH,), jnp.float32),
            pltpu.VMEM((_BLK,), jnp.int32),
            pltpu.VMEM((_BLK,), jnp.int32),
            pltpu.VMEM((_BLK,), jnp.float32),
            pltpu.VMEM((400, 4), jnp.float32),
            pltpu.VMEM((_B, _D), jnp.float32),
            pltpu.VMEM((_B, _D), jnp.float32),
            pltpu.VMEM((_B, _D), jnp.float32),
            pltpu.VMEM((_B, _D), jnp.float32),
            pltpu.VMEM((720,), jnp.float32),
            pltpu.VMEM((720,), jnp.float32),
            pltpu.VMEM((720,), jnp.int32),
            pltpu.VMEM((720,), jnp.int32),
            pltpu.VMEM((_ZCH,), jnp.float32),
            pltpu.SemaphoreType.DMA,
            pltpu.SemaphoreType.DMA,
            pltpu.SemaphoreType.DMA,
            pltpu.SemaphoreType.DMA,
        ],
        compiler_params=pltpu.CompilerParams(needs_layout_passes=False),
    )
    def body(q_hbm, k_hbm, src_hbm, dst_hbm, ea_hbm, out_hbm,
             shared, src2k, dst2k, gate2k, eatmp,
             qb0, qb1, kb0, kb1, sb0, sb1, ib0, ib1, zbuf,
             semr0, semr1, sems0, sems1):
        cidx = lax.axis_index("c")
        sidx = lax.axis_index("s")
        wid = sidx * 2 + cidx
        lanes = lax.iota(jnp.int32, 16)
        rot = [lax.bitwise_and(lanes + d, 15) for d in range(8)]
        zeros16 = jnp.zeros((16,), jnp.float32)
        ones16 = jnp.full((16,), 1.0, jnp.float32)

        qb = (qb0, qb1)
        kb = (kb0, kb1)
        sb = (sb0, sb1)
        ib = (ib0, ib1)
        semr = (semr0, semr1)
        sems = (sems0, sems1)

        for j in range(_ZCH // 16):
            zbuf[pl.ds(j * 16, 16)] = zeros16
        zbase = sidx * (_ZCH * 8)
        for j in range(8):
            pltpu.sync_copy(zbuf, shared.at[pl.ds(zbase + j * _ZCH, _ZCH)])
        plsc.subcore_barrier()

        ebase = wid * _EPW

        def stage_block(abase):
            abase = pl.multiple_of(abase, _B)
            pltpu.sync_copy(src_hbm.at[pl.ds(abase, _BLK)], src2k)
            pltpu.sync_copy(dst_hbm.at[pl.ds(abase, _BLK)], dst2k)
            for p in range(5):
                pltpu.sync_copy(ea_hbm.at[pl.ds(abase + p * 400, 400)], eatmp)

                def gate_body(j, c):
                    joff = pl.multiple_of(j * 16, 16)
                    rowv = j * 16 + lanes
                    asum = zeros16
                    for a in range(4):
                        ac = jnp.full((16,), a, jnp.int32)
                        asum = asum + plsc.load_gather(eatmp, [rowv, ac])
                    gate2k[pl.ds(p * 400 + joff, 16)] = \
                        _SCALE / (1.0 + jnp.exp(-asum))
                    return c

                lax.fori_loop(0, 25, gate_body, 0)

        def issue_rows(off, b):
            off = pl.multiple_of(off, _B)
            pltpu.async_copy(q_hbm.at[src2k.at[pl.ds(off, _B)]], qb[b],
                             semr[b])
            pltpu.async_copy(k_hbm.at[dst2k.at[pl.ds(off, _B)]], kb[b],
                             semr[b])

        def wait_rows(off, b):
            off = pl.multiple_of(off, _B)
            pltpu.make_async_copy(q_hbm.at[src2k.at[pl.ds(off, _B)]], qb[b],
                                  semr[b]).wait()
            pltpu.make_async_copy(k_hbm.at[dst2k.at[pl.ds(off, _B)]], kb[b],
                                  semr[b]).wait()

        def wait_scatter(b):
            pltpu.make_async_copy(sb[b], shared.at[ib[b]], sems[b]).wait()

        def compute_chunk(boff, b):
            qrows = qb[b]
            krows = kb[b]
            scores = sb[b]
            sidxb = ib[b]

            boff16 = pl.multiple_of(boff, 16)

            def group_body(g, c):
                go = pl.multiple_of(g * 16, 16)
                rowv = go + lanes
                dstg = dst2k[pl.ds(boff16 + go, 16)]
                wg = gate2k[pl.ds(boff16 + go, 16)]
                s9i = dstg * 9
                p0 = pl.multiple_of(g * 144, 16)
                for h in range(_H):
                    acc = zeros16
                    hbase = h * 16
                    for d in range(16):
                        if d < 8:
                            colv = rot[d] + hbase
                        else:
                            colv = lax.bitwise_xor(rot[d - 8], 8) + hbase
                        qv = plsc.load_gather(qrows, [rowv, colv])
                        kv = plsc.load_gather(krows, [rowv, colv])
                        acc = acc + qv * kv
                    scores[pl.ds(p0 + h * 16, 16)] = acc * wg
                    sidxb[pl.ds(p0 + h * 16, 16)] = s9i + h
                scores[pl.ds(p0 + 128, 16)] = ones16
                sidxb[pl.ds(p0 + 128, 16)] = s9i + 8
                return c

            lax.fori_loop(0, _B // 16, group_body, 0)

        stage_block(ebase)
        issue_rows(0, 0)

        def outer(t, carry):
            o_cur, a_cur = carry
            for b in range(2):
                o_next = jnp.where(o_cur + _B == _BLK, 0, o_cur + _B)
                a_next = a_cur + _B
                wait_rows(o_cur, b)

                @pl.when(o_next != 0)
                def _():
                    issue_rows(o_next, 1 - b)

                @pl.when(a_cur >= ebase + 2 * _B)
                def _():
                    wait_scatter(b)

                compute_chunk(o_cur, b)
                pltpu.async_copy(sb[b], shared.at[ib[b]], sems[b], add=True)

                @pl.when(o_next == 0)
                def _():
                    stage_block(a_next)
                    issue_rows(o_next, 1 - b)

                o_cur, a_cur = o_next, a_next
            return (o_cur, a_cur)

        o_l, a_l = lax.fori_loop(0, (_NCHUNK - 1) // 2, outer,
                                 (jnp.int32(0), ebase))

        wait_rows(o_l, 0)
        wait_scatter(0)
        compute_chunk(o_l, 0)
        pltpu.async_copy(sb[0], shared.at[ib[0]], sems[0], add=True)
        wait_scatter(1)
        wait_scatter(0)
        plsc.subcore_barrier()

        @pl.when(sidx == 0)
        def _():
            pltpu.sync_copy(shared, out_hbm.at[cidx])

    return body(Q, K, src, dst, edge_attr)


def _combine_body(sp_ref, v_ref, wo_ref, exp_ref, bo_ref, out_ref):
    s_all = jnp.sum(sp_ref[...], axis=0)
    cnt = s_all[:, 8:9]
    s = s_all[:, 0:8]
    m = jnp.where(cnt > 0, s / jnp.maximum(cnt, 1.0), 0.0)
    m128 = lax.dot_general(m, exp_ref[...], (((1,), (0,)), ((), ())),
                           preferred_element_type=jnp.float32)
    dn = (((1,), (1,)), ((), ()))
    out_ref[...] = lax.dot_general(v_ref[...] * m128, wo_ref[...], dn,
                                   preferred_element_type=jnp.float32) \
        + bo_ref[...]


def _combine(sp, V, w_o, b_o):
    blk = 1000
    grid = _N // blk
    return pl.pallas_call(
        _combine_body,
        grid=(grid,),
        in_specs=[
            pl.BlockSpec((2, blk, 9), lambda i: (0, i, 0)),
            pl.BlockSpec((blk, _D), lambda i: (i, 0)),
            pl.BlockSpec((_D, _D), lambda i: (0, 0)),
            pl.BlockSpec((_H, _D), lambda i: (0, 0)),
            pl.BlockSpec((1, _D), lambda i: (0, 0)),
        ],
        out_specs=pl.BlockSpec((blk, _D), lambda i: (i, 0)),
        out_shape=jax.ShapeDtypeStruct((_N, _D), jnp.float32),
    )(sp, V, w_o, _EXPAND, b_o)


def kernel(x, edge_index, edge_attr, w_q, w_k, w_v, w_o, b_o):
    Q, K, V = _projections(x, w_q, w_k, w_v)
    src = edge_index[0]
    dst = edge_index[1]
    sp = _sc_scores(Q, K, src, dst, edge_attr)
    sp9 = sp[:, :9 * _N].reshape(2, _N, 9)
    return _combine(sp9, V, w_o, b_o.reshape(1, _D))
